# baseline (speedup 1.0000x reference)
import functools

import jax
import jax.numpy as jnp
from jax import lax
from jax.experimental import pallas as pl
from jax.experimental.pallas import tpu as pltpu
from jax.experimental.pallas import tpu_sc as plsc

_NUM_CORES = 2
_NUM_SUBCORES = 16
_CHUNK_ROWS = 16
_LANES = 16


def kernel(x, pe):
    batch, seq_len, d_model = x.shape
    n_rows = batch * seq_len
    n_workers = _NUM_CORES * _NUM_SUBCORES
    s_per_w = seq_len // n_workers
    n_chunks = s_per_w // _CHUNK_ROWS
    vregs_per_row = d_model // _LANES

    xf = x.reshape(n_rows, d_model)

    mesh = plsc.VectorSubcoreMesh(core_axis_name="c", subcore_axis_name="s")

    row_chunk = pltpu.VMEM((_CHUNK_ROWS, d_model), jnp.float32)

    @functools.partial(
        pl.kernel,
        mesh=mesh,
        out_type=jax.ShapeDtypeStruct((n_rows, d_model), x.dtype),
        scratch_types=(
            [row_chunk] * batch
            + [row_chunk] * 2
            + [pltpu.SemaphoreType.DMA] * batch
            + [pltpu.SemaphoreType.DMA] * 2
            + [pltpu.SemaphoreType.DMA] * batch
        ),
    )
    def sc_add(x_hbm, pe_hbm, out_hbm, *refs):
        xb = refs[0:batch]
        pb = refs[batch:batch + 2]
        xs = refs[batch + 2:2 * batch + 2]
        ps = refs[2 * batch + 2:2 * batch + 4]
        osem = refs[2 * batch + 4:3 * batch + 4]

        wid = lax.axis_index("s") * _NUM_CORES + lax.axis_index("c")
        s_base = wid * s_per_w

        def x_src(j, b):
            return x_hbm.at[pl.ds(b * seq_len + s_base + j * _CHUNK_ROWS,
                                  _CHUNK_ROWS)]

        def pe_src(j):
            return pe_hbm.at[pl.ds(s_base + j * _CHUNK_ROWS, _CHUNK_ROWS)]

        def out_dst(j, b):
            return out_hbm.at[pl.ds(b * seq_len + s_base + j * _CHUNK_ROWS,
                                    _CHUNK_ROWS)]

        n_items = n_chunks * batch

        for p in range(2):
            pltpu.async_copy(pe_src(p), pb[p], ps[p])
            pltpu.async_copy(x_src(0, p), xb[p], xs[p])

        def item(j, b, pslot):
            i = j * batch + b
            if b == 0:
                pltpu.make_async_copy(pe_src(j), pb[pslot], ps[pslot]).wait()
            pltpu.make_async_copy(x_src(j, b), xb[b], xs[b]).wait()

            def add_body(r, c):
                for k in range(vregs_per_row):
                    sl = pl.ds(k * _LANES, _LANES)
                    plsc.addupdate(xb[b].at[r, sl], pb[pslot][r, sl])
                return c

            lax.fori_loop(0, _CHUNK_ROWS, add_body, 0)

            pltpu.async_copy(xb[b], out_dst(j, b), osem[b])

            bn = (b + 2) % batch
            jd = j - 1 if b < 2 else j
            jn = j if b < 2 else j + 1

            @pl.when(i >= 2)
            def _():
                pltpu.make_async_copy(xb[bn], out_dst(jd, bn),
                                      osem[bn]).wait()

            @pl.when(i + 2 < n_items)
            def _():
                pltpu.async_copy(x_src(jn, bn), xb[bn], xs[bn])

            if b == batch - 1:
                @pl.when(j + 2 < n_chunks)
                def _():
                    pltpu.async_copy(pe_src(j + 2), pb[pslot], ps[pslot])

        def outer(j2, carry):
            for jo in range(2):
                for b in range(batch):
                    item(j2 * 2 + jo, b, pslot=jo)
            return carry

        lax.fori_loop(0, n_chunks // 2, outer, 0)

        for b in range(batch - 2, batch):
            pltpu.make_async_copy(xb[b], out_dst(n_chunks - 1, b),
                                  osem[b]).wait()

    out = sc_add(xf, pe)
    return out.reshape(batch, seq_len, d_model)


# speedup vs baseline: 1.7884x; 1.7884x over previous
import functools

import jax
import jax.numpy as jnp
from jax import lax
from jax.experimental import pallas as pl
from jax.experimental.pallas import tpu as pltpu
from jax.experimental.pallas import tpu_sc as plsc

_NUM_CORES = 2
_NUM_SUBCORES = 16
_CHUNK_ROWS = 16
_LANES = 16


def kernel(x, pe):
    batch, seq_len, d_model = x.shape
    n_rows = batch * seq_len
    n_workers = _NUM_CORES * _NUM_SUBCORES
    s_per_w = seq_len // n_workers
    n_chunks = s_per_w // _CHUNK_ROWS
    vregs_per_row = d_model // _LANES

    xf = x.reshape(n_rows, d_model)

    mesh = plsc.VectorSubcoreMesh(core_axis_name="c", subcore_axis_name="s")

    row_chunk = pltpu.VMEM((_CHUNK_ROWS, d_model), jnp.float32)

    @functools.partial(
        pl.kernel,
        mesh=mesh,
        out_type=jax.ShapeDtypeStruct((n_rows, d_model), x.dtype),
        scratch_types=(
            [row_chunk] * batch
            + [row_chunk] * 2
            + [pltpu.SemaphoreType.DMA] * batch
            + [pltpu.SemaphoreType.DMA] * 2
            + [pltpu.SemaphoreType.DMA] * batch
        ),
    )
    def sc_add(x_hbm, pe_hbm, out_hbm, *refs):
        xb = refs[0:batch]
        pb = refs[batch:batch + 2]
        xs = refs[batch + 2:2 * batch + 2]
        ps = refs[2 * batch + 2:2 * batch + 4]
        osem = refs[2 * batch + 4:3 * batch + 4]

        wid = lax.axis_index("s") * _NUM_CORES + lax.axis_index("c")
        s_base = wid * s_per_w

        def x_src(j, b):
            return x_hbm.at[pl.ds(b * seq_len + s_base + j * _CHUNK_ROWS,
                                  _CHUNK_ROWS)]

        def pe_src(j):
            return pe_hbm.at[pl.ds(s_base + j * _CHUNK_ROWS, _CHUNK_ROWS)]

        def out_dst(j, b):
            return out_hbm.at[pl.ds(b * seq_len + s_base + j * _CHUNK_ROWS,
                                    _CHUNK_ROWS)]

        n_items = n_chunks * batch

        for p in range(2):
            pltpu.async_copy(pe_src(p), pb[p], ps[p])
            pltpu.async_copy(x_src(0, p), xb[p], xs[p])

        def item(j, b, pslot):
            i = j * batch + b
            if b == 0:
                pltpu.make_async_copy(pe_src(j), pb[pslot], ps[pslot]).wait()
            pltpu.make_async_copy(x_src(j, b), xb[b], xs[b]).wait()

            def add_body(r, c):
                for k in range(vregs_per_row):
                    sl = pl.ds(k * _LANES, _LANES)
                    xb[b][r, sl] = xb[b][r, sl] + pb[pslot][r, sl]
                return c

            lax.fori_loop(0, _CHUNK_ROWS, add_body, 0)

            pltpu.async_copy(xb[b], out_dst(j, b), osem[b])

            bn = (b + 2) % batch
            jd = j - 1 if b < 2 else j
            jn = j if b < 2 else j + 1

            @pl.when(i >= 2)
            def _():
                pltpu.make_async_copy(xb[bn], out_dst(jd, bn),
                                      osem[bn]).wait()

            @pl.when(i + 2 < n_items)
            def _():
                pltpu.async_copy(x_src(jn, bn), xb[bn], xs[bn])

            if b == batch - 1:
                @pl.when(j + 2 < n_chunks)
                def _():
                    pltpu.async_copy(pe_src(j + 2), pb[pslot], ps[pslot])

        def outer(j2, carry):
            for jo in range(2):
                for b in range(batch):
                    item(j2 * 2 + jo, b, pslot=jo)
            return carry

        lax.fori_loop(0, n_chunks // 2, outer, 0)

        for b in range(batch - 2, batch):
            pltpu.make_async_copy(xb[b], out_dst(n_chunks - 1, b),
                                  osem[b]).wait()

    out = sc_add(xf, pe)
    return out.reshape(batch, seq_len, d_model)


# speedup vs baseline: 1.8358x; 1.0265x over previous
import functools

import jax
import jax.numpy as jnp
from jax import lax
from jax.experimental import pallas as pl
from jax.experimental.pallas import tpu as pltpu
from jax.experimental.pallas import tpu_sc as plsc

_NUM_CORES = 2
_NUM_SUBCORES = 16
_CHUNK_ROWS = 8
_LANES = 16


def kernel(x, pe):
    batch, seq_len, d_model = x.shape
    n_rows = batch * seq_len
    n_workers = _NUM_CORES * _NUM_SUBCORES
    s_per_w = seq_len // n_workers
    n_chunks = s_per_w // _CHUNK_ROWS
    vregs_per_row = d_model // _LANES

    xf = x.reshape(n_rows, d_model)

    mesh = plsc.VectorSubcoreMesh(core_axis_name="c", subcore_axis_name="s")

    row_chunk = pltpu.VMEM((_CHUNK_ROWS, d_model), jnp.float32)

    @functools.partial(
        pl.kernel,
        mesh=mesh,
        out_type=jax.ShapeDtypeStruct((n_rows, d_model), x.dtype),
        scratch_types=(
            [row_chunk] * batch
            + [row_chunk] * 2
            + [row_chunk] * batch
            + [pltpu.SemaphoreType.DMA] * batch
            + [pltpu.SemaphoreType.DMA] * 2
            + [pltpu.SemaphoreType.DMA] * batch
        ),
    )
    def sc_add(x_hbm, pe_hbm, out_hbm, *refs):
        xb = refs[0:batch]
        pb = refs[batch:batch + 2]
        ob = refs[batch + 2:2 * batch + 2]
        xs = refs[2 * batch + 2:3 * batch + 2]
        ps = refs[3 * batch + 2:3 * batch + 4]
        osem = refs[3 * batch + 4:4 * batch + 4]

        wid = lax.axis_index("s") * _NUM_CORES + lax.axis_index("c")
        s_base = wid * s_per_w

        def x_src(j, b):
            return x_hbm.at[pl.ds(b * seq_len + s_base + j * _CHUNK_ROWS,
                                  _CHUNK_ROWS)]

        def pe_src(j):
            return pe_hbm.at[pl.ds(s_base + j * _CHUNK_ROWS, _CHUNK_ROWS)]

        def out_dst(j, b):
            return out_hbm.at[pl.ds(b * seq_len + s_base + j * _CHUNK_ROWS,
                                    _CHUNK_ROWS)]

        for p in range(2):
            pltpu.async_copy(pe_src(p), pb[p], ps[p])
        for b in range(batch):
            pltpu.async_copy(x_src(0, b), xb[b], xs[b])

        def chunk_body(j, pslot):
            pltpu.make_async_copy(pe_src(j), pb[pslot], ps[pslot]).wait()
            for b in range(batch):
                pltpu.make_async_copy(x_src(j, b), xb[b], xs[b]).wait()

                @pl.when(j >= 1)
                def _():
                    pltpu.make_async_copy(ob[b], out_dst(j - 1, b),
                                          osem[b]).wait()

                def add_body(r, c):
                    for k in range(vregs_per_row):
                        sl = pl.ds(k * _LANES, _LANES)
                        ob[b][r, sl] = xb[b][r, sl] + pb[pslot][r, sl]
                    return c

                lax.fori_loop(0, _CHUNK_ROWS, add_body, 0)

                @pl.when(j + 1 < n_chunks)
                def _():
                    pltpu.async_copy(x_src(j + 1, b), xb[b], xs[b])

                if b == batch - 1:
                    @pl.when(j + 2 < n_chunks)
                    def _():
                        pltpu.async_copy(pe_src(j + 2), pb[pslot], ps[pslot])

                pltpu.async_copy(ob[b], out_dst(j, b), osem[b])

        def outer(j2, carry):
            for jo in range(2):
                chunk_body(j2 * 2 + jo, jo)
            return carry

        lax.fori_loop(0, n_chunks // 2, outer, 0)

        for b in range(batch):
            pltpu.make_async_copy(ob[b], out_dst(n_chunks - 1, b),
                                  osem[b]).wait()

    out = sc_add(xf, pe)
    return out.reshape(batch, seq_len, d_model)
